# baseline (speedup 1.0000x reference)
import functools

import jax
import jax.numpy as jnp
from jax import lax
from jax.experimental import pallas as pl
from jax.experimental.pallas import tpu as pltpu
from jax.experimental.pallas import tpu_sc as plsc

MODEL_DIM = 128
STEP_DIM = MODEL_DIM // 2
SIGNAL_DIM = MODEL_DIM - STEP_DIM
BATCH = 16384

_NC, _NS = 2, 16
_NW = _NC * _NS
_BPW = BATCH // _NW


def _gather_kernel(step_idx_hbm, sig_idx_hbm, step_tab_hbm, sig_tab_hbm,
                   out_hbm, step_idx_v, sig_idx_v, step_rows_v, sig_rows_v,
                   sem0, sem1):
    wid = lax.axis_index("s") * _NC + lax.axis_index("c")
    base = wid * _BPW
    pltpu.sync_copy(step_idx_hbm.at[pl.ds(base, _BPW)], step_idx_v)
    pltpu.sync_copy(sig_idx_hbm.at[pl.ds(base, _BPW)], sig_idx_v)
    cp0 = pltpu.async_copy(step_tab_hbm.at[step_idx_v], step_rows_v, sem0)
    cp1 = pltpu.async_copy(sig_tab_hbm.at[sig_idx_v], sig_rows_v, sem1)
    cp0.wait()
    pltpu.sync_copy(step_rows_v, out_hbm.at[pl.ds(base, _BPW), 0])
    cp1.wait()
    pltpu.sync_copy(sig_rows_v, out_hbm.at[pl.ds(base, _BPW), 1])


@jax.jit
def _lookup(step_idx, sig_idx, step_tab, sig_tab):
    k = functools.partial(
        pl.kernel,
        out_type=jax.ShapeDtypeStruct((BATCH, 2, STEP_DIM), jnp.float32),
        mesh=plsc.VectorSubcoreMesh(core_axis_name="c", subcore_axis_name="s"),
        compiler_params=pltpu.CompilerParams(use_tc_tiling_on_sc=False),
        scratch_types=[
            pltpu.VMEM((_BPW,), jnp.int32),
            pltpu.VMEM((_BPW,), jnp.int32),
            pltpu.VMEM((_BPW, STEP_DIM), jnp.float32),
            pltpu.VMEM((_BPW, SIGNAL_DIM), jnp.float32),
            pltpu.SemaphoreType.DMA,
            pltpu.SemaphoreType.DMA,
        ],
    )(_gather_kernel)
    return k(step_idx, sig_idx, step_tab, sig_tab)


def kernel(step_levels, signal_levels, step_embedding, signal_embedding):
    step_idx = jnp.asarray(step_levels, dtype=jnp.int32)
    sig_idx = jnp.asarray(signal_levels, dtype=jnp.int32)
    out = _lookup(step_idx, sig_idx, step_embedding, signal_embedding)
    return out.reshape(BATCH, MODEL_DIM).astype(jnp.bfloat16)
